# Initial kernel scaffold; baseline (speedup 1.0000x reference)
import jax
import jax.numpy as jnp
from jax.experimental import pallas as pl


def kernel(x, edge_index, trn_w, trn_b, prd_w, prd_b):
    raise NotImplementedError("write your pallas kernel here")


# speedup vs baseline: 4.1700x; 4.1700x over previous
import functools

import jax
import jax.numpy as jnp
from jax import lax
from jax.experimental import pallas as pl
from jax.experimental.pallas import tpu as pltpu
from jax.experimental.pallas import tpu_sc as plsc

N_NODES = 10000
N_EDGES = 320000
HIDDEN = 128
MIDDLE = 320
DICT = 512

NUM_CORES = 2
NUM_SUBCORES = 16
HALF = N_NODES // NUM_CORES
ACC_ROWS = HALF + NUM_SUBCORES
EDGES_PER_TILE = N_EDGES // NUM_SUBCORES
BATCH = 80
ITERS = EDGES_PER_TILE // BATCH
LANES = 16
STRIPE = (HALF // NUM_SUBCORES) // 8 * 8
W_TAIL = HALF - NUM_SUBCORES * STRIPE
Z_TAIL = ACC_ROWS - NUM_SUBCORES * STRIPE
ZROWS = 104


@functools.partial(
    pl.kernel,
    mesh=plsc.VectorSubcoreMesh(core_axis_name="c", subcore_axis_name="s"),
    out_type=jax.ShapeDtypeStruct((NUM_CORES, HALF, HIDDEN), jnp.float32),
    scratch_types=[
        pltpu.VMEM((ITERS, BATCH), jnp.int32),
        pltpu.VMEM((ITERS, BATCH), jnp.int32),
        pltpu.VMEM((BATCH, HIDDEN), jnp.float32),
        pltpu.VMEM((ZROWS, HIDDEN), jnp.float32),
        pltpu.VMEM_SHARED((ACC_ROWS, HIDDEN), jnp.float32),
        pltpu.SemaphoreType.DMA,
    ],
)
def _sc_agg(x_hbm, src_hbm, dst_hbm, out_hbm, src_v, dst_v, rows_v, zbuf, acc, sem):
    c = lax.axis_index("c")
    s = lax.axis_index("s")

    def _zero_body(i, carry):
        for t in range(HIDDEN // LANES):
            zbuf[i, pl.ds(t * LANES, LANES)] = jnp.zeros((LANES,), jnp.float32)
        return carry

    lax.fori_loop(0, ZROWS, _zero_body, 0)
    for k in range(STRIPE // ZROWS):
        pltpu.sync_copy(zbuf, acc.at[pl.ds(s * STRIPE + k * ZROWS, ZROWS)])

    @pl.when(s == 0)
    def _zero_tail():
        pltpu.sync_copy(
            zbuf.at[pl.ds(0, Z_TAIL)],
            acc.at[pl.ds(NUM_SUBCORES * STRIPE, Z_TAIL)],
        )

    pltpu.sync_copy(src_hbm.at[s], src_v)
    pltpu.sync_copy(dst_hbm.at[s], dst_v)

    lo = c * HALF
    trash = HALF + s
    def _remap_body(i, carry):
        for t in range(BATCH // LANES):
            v = dst_v[i, pl.ds(t * LANES, LANES)]
            rel = v - lo
            inb = (rel >= 0) & (rel < HALF)
            dst_v[i, pl.ds(t * LANES, LANES)] = jnp.where(inb, rel, trash)
        return carry

    lax.fori_loop(0, ITERS, _remap_body, 0)
    plsc.subcore_barrier()

    def _edge_body(j, carry):
        pltpu.async_copy(x_hbm.at[src_v.at[j]], rows_v, sem).wait()
        pltpu.sync_copy(rows_v, acc.at[dst_v.at[j]], add=True)
        return carry

    lax.fori_loop(0, ITERS, _edge_body, 0)
    plsc.subcore_barrier()

    pltpu.sync_copy(
        acc.at[pl.ds(s * STRIPE, STRIPE)],
        out_hbm.at[c, pl.ds(s * STRIPE, STRIPE)],
    )

    @pl.when(s == 0)
    def _write_tail():
        pltpu.sync_copy(
            acc.at[pl.ds(NUM_SUBCORES * STRIPE, W_TAIL)],
            out_hbm.at[c, pl.ds(NUM_SUBCORES * STRIPE, W_TAIL)],
        )


def _mlp_body(x_ref, a_ref, tw_ref, tb_ref, pw_ref, pb_ref, o_ref):
    h = x_ref[...] + a_ref[...]
    h1 = jnp.dot(h, tw_ref[...], preferred_element_type=jnp.float32) + tb_ref[...]
    o_ref[...] = jnp.dot(h1, pw_ref[...], preferred_element_type=jnp.float32) + pb_ref[...]


ROW_BLOCK = 1000


def _tc_mlp(x, agg, trn_w, trn_b, prd_w, prd_b):
    return pl.pallas_call(
        _mlp_body,
        grid=(N_NODES // ROW_BLOCK,),
        in_specs=[
            pl.BlockSpec((ROW_BLOCK, HIDDEN), lambda i: (i, 0)),
            pl.BlockSpec((ROW_BLOCK, HIDDEN), lambda i: (i, 0)),
            pl.BlockSpec((HIDDEN, MIDDLE), lambda i: (0, 0)),
            pl.BlockSpec((1, MIDDLE), lambda i: (0, 0)),
            pl.BlockSpec((MIDDLE, DICT), lambda i: (0, 0)),
            pl.BlockSpec((1, DICT), lambda i: (0, 0)),
        ],
        out_specs=pl.BlockSpec((ROW_BLOCK, DICT), lambda i: (i, 0)),
        out_shape=jax.ShapeDtypeStruct((N_NODES, DICT), jnp.float32),
    )(x, agg, trn_w, trn_b, prd_w, prd_b)


def kernel(x, edge_index, trn_w, trn_b, prd_w, prd_b):
    ei = edge_index.astype(jnp.int32)
    src = ei[0].reshape(NUM_SUBCORES, ITERS, BATCH)
    dst = ei[1].reshape(NUM_SUBCORES, ITERS, BATCH)
    agg = _sc_agg(x, src, dst).reshape(N_NODES, HIDDEN)
    return _tc_mlp(
        x,
        agg,
        trn_w,
        trn_b.reshape(1, MIDDLE),
        prd_w,
        prd_b.reshape(1, DICT),
    )


# speedup vs baseline: 6.9092x; 1.6569x over previous
import functools

import jax
import jax.numpy as jnp
from jax import lax
from jax.experimental import pallas as pl
from jax.experimental.pallas import tpu as pltpu
from jax.experimental.pallas import tpu_sc as plsc

N_NODES = 10000
N_EDGES = 320000
HIDDEN = 128
MIDDLE = 320
DICT = 512

NUM_CORES = 2
NUM_SUBCORES = 16
HALF = N_NODES // NUM_CORES
ACC_ROWS = HALF + NUM_SUBCORES
EDGES_PER_TILE = N_EDGES // NUM_SUBCORES
BATCH = 80
ITERS = EDGES_PER_TILE // BATCH
LANES = 16
STRIPE = (HALF // NUM_SUBCORES) // 8 * 8
W_TAIL = HALF - NUM_SUBCORES * STRIPE
Z_TAIL = ACC_ROWS - NUM_SUBCORES * STRIPE
ZROWS = 24


@functools.partial(
    pl.kernel,
    mesh=plsc.VectorSubcoreMesh(core_axis_name="c", subcore_axis_name="s"),
    out_type=jax.ShapeDtypeStruct((NUM_CORES, HALF, HIDDEN), jnp.float32),
    scratch_types=[
        pltpu.VMEM((ITERS, BATCH), jnp.int32),
        pltpu.VMEM((ITERS, BATCH), jnp.int32),
        pltpu.VMEM((BATCH, HIDDEN), jnp.float32),
        pltpu.VMEM((BATCH, HIDDEN), jnp.float32),
        pltpu.VMEM((ZROWS, HIDDEN), jnp.float32),
        pltpu.VMEM_SHARED((ACC_ROWS, HIDDEN), jnp.float32),
        pltpu.SemaphoreType.DMA,
        pltpu.SemaphoreType.DMA,
    ],
)
def _sc_agg(x_hbm, src_hbm, dst_hbm, out_hbm, src_v, dst_v, rows_a, rows_b,
            zbuf, acc, sem_a, sem_b):
    c = lax.axis_index("c")
    s = lax.axis_index("s")

    def _zero_body(i, carry):
        for t in range(HIDDEN // LANES):
            zbuf[i, pl.ds(t * LANES, LANES)] = jnp.zeros((LANES,), jnp.float32)
        return carry

    lax.fori_loop(0, ZROWS, _zero_body, 0)
    for k in range(STRIPE // ZROWS):
        pltpu.sync_copy(zbuf, acc.at[pl.ds(s * STRIPE + k * ZROWS, ZROWS)])

    @pl.when(s == 0)
    def _zero_tail():
        pltpu.sync_copy(
            zbuf.at[pl.ds(0, Z_TAIL)],
            acc.at[pl.ds(NUM_SUBCORES * STRIPE, Z_TAIL)],
        )

    pltpu.sync_copy(src_hbm.at[s], src_v)
    pltpu.sync_copy(dst_hbm.at[s], dst_v)

    lo = c * HALF
    trash = HALF + s
    def _remap_body(i, carry):
        for t in range(BATCH // LANES):
            v = dst_v[i, pl.ds(t * LANES, LANES)]
            rel = v - lo
            inb = (rel >= 0) & (rel < HALF)
            dst_v[i, pl.ds(t * LANES, LANES)] = jnp.where(inb, rel, trash)
        return carry

    lax.fori_loop(0, ITERS, _remap_body, 0)
    plsc.subcore_barrier()

    pltpu.async_copy(x_hbm.at[src_v.at[0]], rows_a, sem_a)

    def _edge_pair(i, carry):
        j = 2 * i
        pltpu.async_copy(x_hbm.at[src_v.at[j + 1]], rows_b, sem_b)
        pltpu.make_async_copy(x_hbm.at[src_v.at[j]], rows_a, sem_a).wait()
        pltpu.sync_copy(rows_a, acc.at[dst_v.at[j]], add=True)

        @pl.when(j + 2 < ITERS)
        def _():
            pltpu.async_copy(x_hbm.at[src_v.at[j + 2]], rows_a, sem_a)

        pltpu.make_async_copy(x_hbm.at[src_v.at[j + 1]], rows_b, sem_b).wait()
        pltpu.sync_copy(rows_b, acc.at[dst_v.at[j + 1]], add=True)
        return carry

    lax.fori_loop(0, ITERS // 2, _edge_pair, 0)
    plsc.subcore_barrier()

    pltpu.sync_copy(
        acc.at[pl.ds(s * STRIPE, STRIPE)],
        out_hbm.at[c, pl.ds(s * STRIPE, STRIPE)],
    )

    @pl.when(s == 0)
    def _write_tail():
        pltpu.sync_copy(
            acc.at[pl.ds(NUM_SUBCORES * STRIPE, W_TAIL)],
            out_hbm.at[c, pl.ds(NUM_SUBCORES * STRIPE, W_TAIL)],
        )


def _mlp_body(x_ref, a_ref, tw_ref, tb_ref, pw_ref, pb_ref, o_ref):
    h = x_ref[...] + a_ref[...]
    h1 = jnp.dot(h, tw_ref[...], preferred_element_type=jnp.float32) + tb_ref[...]
    o_ref[...] = jnp.dot(h1, pw_ref[...], preferred_element_type=jnp.float32) + pb_ref[...]


ROW_BLOCK = 1000


def _tc_mlp(x, agg, trn_w, trn_b, prd_w, prd_b):
    return pl.pallas_call(
        _mlp_body,
        grid=(N_NODES // ROW_BLOCK,),
        in_specs=[
            pl.BlockSpec((ROW_BLOCK, HIDDEN), lambda i: (i, 0)),
            pl.BlockSpec((ROW_BLOCK, HIDDEN), lambda i: (i, 0)),
            pl.BlockSpec((HIDDEN, MIDDLE), lambda i: (0, 0)),
            pl.BlockSpec((1, MIDDLE), lambda i: (0, 0)),
            pl.BlockSpec((MIDDLE, DICT), lambda i: (0, 0)),
            pl.BlockSpec((1, DICT), lambda i: (0, 0)),
        ],
        out_specs=pl.BlockSpec((ROW_BLOCK, DICT), lambda i: (i, 0)),
        out_shape=jax.ShapeDtypeStruct((N_NODES, DICT), jnp.float32),
    )(x, agg, trn_w, trn_b, prd_w, prd_b)


def kernel(x, edge_index, trn_w, trn_b, prd_w, prd_b):
    ei = edge_index.astype(jnp.int32)
    src = ei[0].reshape(NUM_SUBCORES, ITERS, BATCH)
    dst = ei[1].reshape(NUM_SUBCORES, ITERS, BATCH)
    agg = _sc_agg(x, src, dst).reshape(N_NODES, HIDDEN)
    return _tc_mlp(
        x,
        agg,
        trn_w,
        trn_b.reshape(1, MIDDLE),
        prd_w,
        prd_b.reshape(1, DICT),
    )


# speedup vs baseline: 7.6947x; 1.1137x over previous
import functools

import jax
import jax.numpy as jnp
from jax import lax
from jax.experimental import pallas as pl
from jax.experimental.pallas import tpu as pltpu
from jax.experimental.pallas import tpu_sc as plsc

N_NODES = 10000
N_EDGES = 320000
HIDDEN = 128
MIDDLE = 320
DICT = 512

NUM_CORES = 2
NUM_SUBCORES = 16
NUM_WORKERS = NUM_CORES * NUM_SUBCORES
BATCH = 40
BPC = 25
CHUNKS = N_EDGES // (NUM_WORKERS * BPC * BATCH)
STRIPE = 624
TAIL = N_NODES - NUM_SUBCORES * STRIPE


@functools.partial(
    pl.kernel,
    mesh=plsc.VectorSubcoreMesh(core_axis_name="c", subcore_axis_name="s"),
    out_type=jax.ShapeDtypeStruct((NUM_CORES, N_NODES, HIDDEN), jnp.float32),
    scratch_types=[
        pltpu.VMEM((BPC, BATCH), jnp.int32),
        pltpu.VMEM((BPC, BATCH), jnp.int32),
        pltpu.VMEM((BATCH, HIDDEN), jnp.float32),
        pltpu.VMEM((BATCH, HIDDEN), jnp.float32),
        pltpu.VMEM_SHARED((N_NODES, HIDDEN), jnp.float32),
        pltpu.SemaphoreType.DMA,
        pltpu.SemaphoreType.DMA,
    ],
)
def _sc_agg(x_hbm, src_hbm, dst_hbm, zeros_hbm, out_hbm, src_v, dst_v,
            rows_a, rows_b, acc, sem_a, sem_b):
    c = lax.axis_index("c")
    s = lax.axis_index("s")
    wid = s * NUM_CORES + c

    pltpu.sync_copy(zeros_hbm, acc.at[pl.ds(s * STRIPE, STRIPE)])

    @pl.when(s == 0)
    def _zero_tail():
        pltpu.sync_copy(
            zeros_hbm.at[pl.ds(0, TAIL)],
            acc.at[pl.ds(NUM_SUBCORES * STRIPE, TAIL)],
        )

    plsc.subcore_barrier()

    def _chunk(k, carry):
        pltpu.sync_copy(src_hbm.at[wid, k], src_v)
        pltpu.sync_copy(dst_hbm.at[wid, k], dst_v)
        pltpu.async_copy(x_hbm.at[src_v.at[0]], rows_a, sem_a)

        def _pair(i, cc):
            j = 2 * i + 1
            pltpu.async_copy(x_hbm.at[src_v.at[j]], rows_b, sem_b)
            pltpu.make_async_copy(x_hbm.at[src_v.at[0]], rows_a, sem_a).wait()
            pltpu.sync_copy(rows_a, acc.at[dst_v.at[j - 1]], add=True)
            pltpu.async_copy(x_hbm.at[src_v.at[j + 1]], rows_a, sem_a)
            pltpu.make_async_copy(x_hbm.at[src_v.at[0]], rows_b, sem_b).wait()
            pltpu.sync_copy(rows_b, acc.at[dst_v.at[j]], add=True)
            return cc

        lax.fori_loop(0, (BPC - 1) // 2, _pair, 0)
        pltpu.make_async_copy(x_hbm.at[src_v.at[0]], rows_a, sem_a).wait()
        pltpu.sync_copy(rows_a, acc.at[dst_v.at[BPC - 1]], add=True)
        return carry

    lax.fori_loop(0, CHUNKS, _chunk, 0)
    plsc.subcore_barrier()

    pltpu.sync_copy(
        acc.at[pl.ds(s * STRIPE, STRIPE)],
        out_hbm.at[c, pl.ds(s * STRIPE, STRIPE)],
    )

    @pl.when(s == 0)
    def _write_tail():
        pltpu.sync_copy(
            acc.at[pl.ds(NUM_SUBCORES * STRIPE, TAIL)],
            out_hbm.at[c, pl.ds(NUM_SUBCORES * STRIPE, TAIL)],
        )


def _mlp_body(x_ref, a0_ref, a1_ref, tw_ref, tb_ref, pw_ref, pb_ref, o_ref):
    h = x_ref[...] + a0_ref[...] + a1_ref[...]
    h1 = jnp.dot(h, tw_ref[...], preferred_element_type=jnp.float32) + tb_ref[...]
    o_ref[...] = jnp.dot(h1, pw_ref[...], preferred_element_type=jnp.float32) + pb_ref[...]


ROW_BLOCK = 1000


def _tc_mlp(x, a0, a1, trn_w, trn_b, prd_w, prd_b):
    return pl.pallas_call(
        _mlp_body,
        grid=(N_NODES // ROW_BLOCK,),
        in_specs=[
            pl.BlockSpec((ROW_BLOCK, HIDDEN), lambda i: (i, 0)),
            pl.BlockSpec((ROW_BLOCK, HIDDEN), lambda i: (i, 0)),
            pl.BlockSpec((ROW_BLOCK, HIDDEN), lambda i: (i, 0)),
            pl.BlockSpec((HIDDEN, MIDDLE), lambda i: (0, 0)),
            pl.BlockSpec((1, MIDDLE), lambda i: (0, 0)),
            pl.BlockSpec((MIDDLE, DICT), lambda i: (0, 0)),
            pl.BlockSpec((1, DICT), lambda i: (0, 0)),
        ],
        out_specs=pl.BlockSpec((ROW_BLOCK, DICT), lambda i: (i, 0)),
        out_shape=jax.ShapeDtypeStruct((N_NODES, DICT), jnp.float32),
    )(x, a0, a1, trn_w, trn_b, prd_w, prd_b)


def kernel(x, edge_index, trn_w, trn_b, prd_w, prd_b):
    ei = edge_index.astype(jnp.int32)
    src = ei[0].reshape(NUM_WORKERS, CHUNKS, BPC, BATCH)
    dst = ei[1].reshape(NUM_WORKERS, CHUNKS, BPC, BATCH)
    zeros = jnp.zeros((STRIPE, HIDDEN), jnp.float32)
    agg = _sc_agg(x, src, dst, zeros)
    return _tc_mlp(
        x,
        agg[0],
        agg[1],
        trn_w,
        trn_b.reshape(1, MIDDLE),
        prd_w,
        prd_b.reshape(1, DICT),
    )


# speedup vs baseline: 9.5751x; 1.2444x over previous
import functools

import jax
import jax.numpy as jnp
from jax import lax
from jax.experimental import pallas as pl
from jax.experimental.pallas import tpu as pltpu
from jax.experimental.pallas import tpu_sc as plsc

N_NODES = 10000
N_EDGES = 320000
HIDDEN = 128
MIDDLE = 320
DICT = 512

NUM_CORES = 2
NUM_SUBCORES = 16
NUM_WORKERS = NUM_CORES * NUM_SUBCORES
BATCH = 40
BPC = 25
CHUNKS = N_EDGES // (NUM_WORKERS * BPC * BATCH)
STRIPE = 624
TAIL = N_NODES - NUM_SUBCORES * STRIPE


@functools.partial(
    pl.kernel,
    mesh=plsc.VectorSubcoreMesh(core_axis_name="c", subcore_axis_name="s"),
    out_type=jax.ShapeDtypeStruct((NUM_CORES, N_NODES, HIDDEN), jnp.float32),
    scratch_types=[
        pltpu.VMEM((BPC, BATCH), jnp.int32),
        pltpu.VMEM((BPC, BATCH), jnp.int32),
        pltpu.VMEM((BATCH, HIDDEN), jnp.float32),
        pltpu.VMEM((BATCH, HIDDEN), jnp.float32),
        pltpu.VMEM((BATCH, HIDDEN), jnp.float32),
        pltpu.VMEM((BATCH, HIDDEN), jnp.float32),
        pltpu.VMEM_SHARED((N_NODES, HIDDEN), jnp.float32),
        pltpu.SemaphoreType.DMA,
        pltpu.SemaphoreType.DMA,
        pltpu.SemaphoreType.DMA,
        pltpu.SemaphoreType.DMA,
        pltpu.SemaphoreType.DMA,
        pltpu.SemaphoreType.DMA,
        pltpu.SemaphoreType.DMA,
        pltpu.SemaphoreType.DMA,
    ],
)
def _sc_agg(x_hbm, src_hbm, dst_hbm, zeros_hbm, out_hbm, src_v, dst_v,
            r0, r1, r2, r3, acc, g0, g1, g2, g3, s0, s1, s2, s3):
    c = lax.axis_index("c")
    s = lax.axis_index("s")
    wid = s * NUM_CORES + c

    pltpu.sync_copy(zeros_hbm, acc.at[pl.ds(s * STRIPE, STRIPE)])

    @pl.when(s == 0)
    def _zero_tail():
        pltpu.sync_copy(
            zeros_hbm.at[pl.ds(0, TAIL)],
            acc.at[pl.ds(NUM_SUBCORES * STRIPE, TAIL)],
        )

    plsc.subcore_barrier()

    rows = (r0, r1, r2, r3)
    gsem = (g0, g1, g2, g3)
    ssem = (s0, s1, s2, s3)

    def _wait_gather(u):
        pltpu.make_async_copy(x_hbm.at[src_v.at[0]], rows[u], gsem[u]).wait()

    def _wait_scatter(u):
        pltpu.make_async_copy(rows[u], acc.at[dst_v.at[0]], ssem[u]).wait()

    def _step(j, issue_j):
        u = j % 4
        _wait_gather(u)
        pltpu.async_copy(rows[u], acc.at[dst_v.at[j]], ssem[u], add=True)
        if issue_j is not None:
            v = issue_j % 4
            if issue_j >= 4:
                _wait_scatter(v)
            pltpu.async_copy(x_hbm.at[src_v.at[issue_j]], rows[v], gsem[v])

    def _chunk(k, carry):
        pltpu.sync_copy(src_hbm.at[wid, k], src_v)
        pltpu.sync_copy(dst_hbm.at[wid, k], dst_v)
        for j in range(3):
            pltpu.async_copy(x_hbm.at[src_v.at[j]], rows[j], gsem[j])

        def _quad(i, cc):
            j0 = 4 * i

            def _dyn_step(u, off):
                _wait_gather(u)
                pltpu.async_copy(rows[u], acc.at[dst_v.at[j0 + u]], ssem[u],
                                 add=True)
                v = (u + 3) % 4
                @pl.when(j0 + u >= 1)
                def _():
                    _wait_scatter(v)
                pltpu.async_copy(x_hbm.at[src_v.at[j0 + u + 3]], rows[v],
                                 gsem[v])

            for u in range(4):
                _dyn_step(u, 0)
            return cc

        lax.fori_loop(0, 5, _quad, 0)
        _step(20, 23)
        _step(21, 24)
        _step(22, None)
        _step(23, None)
        _step(24, None)
        for u in range(4):
            _wait_scatter(u)
        return carry

    lax.fori_loop(0, CHUNKS, _chunk, 0)
    plsc.subcore_barrier()

    pltpu.sync_copy(
        acc.at[pl.ds(s * STRIPE, STRIPE)],
        out_hbm.at[c, pl.ds(s * STRIPE, STRIPE)],
    )

    @pl.when(s == 0)
    def _write_tail():
        pltpu.sync_copy(
            acc.at[pl.ds(NUM_SUBCORES * STRIPE, TAIL)],
            out_hbm.at[c, pl.ds(NUM_SUBCORES * STRIPE, TAIL)],
        )


def _mlp_body(x_ref, a0_ref, a1_ref, tw_ref, tb_ref, pw_ref, pb_ref, o_ref):
    h = x_ref[...] + a0_ref[...] + a1_ref[...]
    h1 = jnp.dot(h, tw_ref[...], preferred_element_type=jnp.float32) + tb_ref[...]
    o_ref[...] = jnp.dot(h1, pw_ref[...], preferred_element_type=jnp.float32) + pb_ref[...]


ROW_BLOCK = 1000


def _tc_mlp(x, a0, a1, trn_w, trn_b, prd_w, prd_b):
    return pl.pallas_call(
        _mlp_body,
        grid=(N_NODES // ROW_BLOCK,),
        in_specs=[
            pl.BlockSpec((ROW_BLOCK, HIDDEN), lambda i: (i, 0)),
            pl.BlockSpec((ROW_BLOCK, HIDDEN), lambda i: (i, 0)),
            pl.BlockSpec((ROW_BLOCK, HIDDEN), lambda i: (i, 0)),
            pl.BlockSpec((HIDDEN, MIDDLE), lambda i: (0, 0)),
            pl.BlockSpec((1, MIDDLE), lambda i: (0, 0)),
            pl.BlockSpec((MIDDLE, DICT), lambda i: (0, 0)),
            pl.BlockSpec((1, DICT), lambda i: (0, 0)),
        ],
        out_specs=pl.BlockSpec((ROW_BLOCK, DICT), lambda i: (i, 0)),
        out_shape=jax.ShapeDtypeStruct((N_NODES, DICT), jnp.float32),
    )(x, a0, a1, trn_w, trn_b, prd_w, prd_b)


def kernel(x, edge_index, trn_w, trn_b, prd_w, prd_b):
    ei = edge_index.astype(jnp.int32)
    src = ei[0].reshape(NUM_WORKERS, CHUNKS, BPC, BATCH)
    dst = ei[1].reshape(NUM_WORKERS, CHUNKS, BPC, BATCH)
    zeros = jnp.zeros((STRIPE, HIDDEN), jnp.float32)
    agg = _sc_agg(x, src, dst, zeros)
    return _tc_mlp(
        x,
        agg[0],
        agg[1],
        trn_w,
        trn_b.reshape(1, MIDDLE),
        prd_w,
        prd_b.reshape(1, DICT),
    )
